# baseline (speedup 1.0000x reference)
import functools

import jax
import jax.numpy as jnp
from jax import lax
from jax.experimental import pallas as pl
from jax.experimental.pallas import tpu as pltpu
from jax.experimental.pallas import tpu_sc as plsc

DIM = 10000
SIZE = 32
LEVELS = 256
BATCH = 64

NW = 32
CPW = 320
DPAD = NW * CPW
XG = 16
NXG = SIZE // XG
NCP = 1
CPASS = CPW // (16 * NCP)


def _sc_encode(img_r, vt_r, xt_r, yt_r):
    mesh = plsc.VectorSubcoreMesh(core_axis_name="c", subcore_axis_name="s")

    @functools.partial(
        pl.kernel,
        mesh=mesh,
        out_type=jax.ShapeDtypeStruct((NW, BATCH, CPW), jnp.float32),
        scratch_types=[
            pltpu.VMEM((LEVELS, CPW), jnp.float32),
            pltpu.VMEM((SIZE, CPW), jnp.float32),
            pltpu.VMEM((CPW,), jnp.float32),
            pltpu.VMEM((BATCH, CPW), jnp.float32),
            pltpu.VMEM((BATCH, SIZE), jnp.int32),
            pltpu.SemaphoreType.DMA,
            pltpu.SemaphoreType.DMA,
        ],
        compiler_params=pltpu.CompilerParams(use_tc_tiling_on_sc=False),
    )
    def enc(img_hbm, vt_hbm, xt_hbm, yt_hbm, out_hbm,
            vt_s, xt_s, yt_row, acc_s, idx_s, sem1, sem2):
        w = lax.axis_index("s") * 2 + lax.axis_index("c")
        cp1 = pltpu.async_copy(vt_hbm.at[w], vt_s, sem1)
        cp2 = pltpu.async_copy(xt_hbm.at[w], xt_s, sem2)

        zero = jnp.zeros((16,), jnp.float32)

        @pl.loop(0, BATCH)
        def _(b):
            for c in range(CPW // 16):
                acc_s[b, pl.ds(c * 16, 16)] = zero

        cp1.wait()
        cp2.wait()

        @pl.loop(0, SIZE)
        def _(y):
            pltpu.sync_copy(img_hbm.at[y], idx_s)
            pltpu.sync_copy(yt_hbm.at[w, y], yt_row)

            @pl.loop(0, CPASS)
            def _(cp):
                cbase = cp * (16 * NCP)
                for xg in range(NXG):
                    wv = [[xt_s[xg * XG + p, pl.ds(cbase + c * 16, 16)]
                           * yt_row[pl.ds(cbase + c * 16, 16)]
                           for c in range(NCP)]
                          for p in range(XG)]

                    @plsc.parallel_loop(0, BATCH)
                    def _(b):
                        lvec = idx_s[b, pl.ds(xg * XG, 16)]
                        for c in range(NCP):
                            acc = None
                            for p in range(XG):
                                term = (vt_s[lvec[p],
                                             pl.ds(cbase + c * 16, 16)]
                                        * wv[p][c])
                                acc = term if acc is None else acc + term
                            plsc.addupdate(
                                acc_s.at[b, pl.ds(cbase + c * 16, 16)], acc)

        one = jnp.full((16,), 1.0, jnp.float32)
        mone = jnp.full((16,), -1.0, jnp.float32)

        @pl.loop(0, BATCH)
        def _(b):
            for c in range(CPW // 16):
                v = acc_s[b, pl.ds(c * 16, 16)]
                acc_s[b, pl.ds(c * 16, 16)] = jnp.where(v > 0.0, one, mone)

        pltpu.sync_copy(acc_s, out_hbm.at[w])

    return enc(img_r, vt_r, xt_r, yt_r)


def kernel(x, value_table, x_table, y_table):
    img = x.reshape(BATCH, SIZE, SIZE).astype(jnp.int32)
    img_r = img.transpose(1, 0, 2)

    pad = DPAD - DIM
    vt = jnp.pad(value_table, ((0, 0), (0, pad)))
    xt = jnp.pad(x_table, ((0, 0), (0, pad)))
    yt = jnp.pad(y_table, ((0, 0), (0, pad)))
    vt_r = vt.reshape(LEVELS, NW, CPW).transpose(1, 0, 2)
    xt_r = xt.reshape(SIZE, NW, CPW).transpose(1, 0, 2)
    yt_r = yt.reshape(SIZE, NW, CPW).transpose(1, 0, 2)

    out = _sc_encode(img_r, vt_r, xt_r, yt_r)
    return out.transpose(1, 0, 2).reshape(BATCH, DPAD)[:, :DIM]


# speedup vs baseline: 1.8327x; 1.8327x over previous
import functools

import jax
import jax.numpy as jnp
from jax import lax
from jax.experimental import pallas as pl
from jax.experimental.pallas import tpu as pltpu
from jax.experimental.pallas import tpu_sc as plsc

DIM = 10000
SIZE = 32
LEVELS = 256
BATCH = 64
DPAD = 10240

NW = 32
CPW = 160
DSC = NW * CPW
XG = 16
NXG = SIZE // XG
NCP = 2
CPASS = CPW // (16 * NCP)

DTC = DPAD - DSC
DBLK = 512


def _sc_encode(img_r, vt_r, xt_r, yt_r):
    mesh = plsc.VectorSubcoreMesh(core_axis_name="c", subcore_axis_name="s")

    @functools.partial(
        pl.kernel,
        mesh=mesh,
        out_type=jax.ShapeDtypeStruct((NW, BATCH, CPW), jnp.float32),
        scratch_types=[
            pltpu.VMEM((LEVELS, CPW), jnp.float32),
            pltpu.VMEM((SIZE, CPW), jnp.float32),
            pltpu.VMEM((CPW,), jnp.float32),
            pltpu.VMEM((BATCH, CPW), jnp.float32),
            pltpu.VMEM((BATCH, SIZE), jnp.int32),
            pltpu.SemaphoreType.DMA,
            pltpu.SemaphoreType.DMA,
        ],
        compiler_params=pltpu.CompilerParams(use_tc_tiling_on_sc=False),
    )
    def enc(img_hbm, vt_hbm, xt_hbm, yt_hbm, out_hbm,
            vt_s, xt_s, yt_row, acc_s, idx_s, sem1, sem2):
        w = lax.axis_index("s") * 2 + lax.axis_index("c")
        cp1 = pltpu.async_copy(vt_hbm.at[w], vt_s, sem1)
        cp2 = pltpu.async_copy(xt_hbm.at[w], xt_s, sem2)

        zero = jnp.zeros((16,), jnp.float32)

        @pl.loop(0, BATCH)
        def _(b):
            for c in range(CPW // 16):
                acc_s[b, pl.ds(c * 16, 16)] = zero

        cp1.wait()
        cp2.wait()

        @pl.loop(0, SIZE)
        def _(y):
            pltpu.sync_copy(img_hbm.at[y], idx_s)
            pltpu.sync_copy(yt_hbm.at[w, y], yt_row)

            @pl.loop(0, CPASS)
            def _(cp):
                cbase = cp * (16 * NCP)
                for xg in range(NXG):
                    wv = [[xt_s[xg * XG + p, pl.ds(cbase + c * 16, 16)]
                           * yt_row[pl.ds(cbase + c * 16, 16)]
                           for c in range(NCP)]
                          for p in range(XG)]

                    @plsc.parallel_loop(0, BATCH)
                    def _(b):
                        lvec = idx_s[b, pl.ds(xg * XG, 16)]
                        for c in range(NCP):
                            acc = None
                            for p in range(XG):
                                term = (vt_s[lvec[p],
                                             pl.ds(cbase + c * 16, 16)]
                                        * wv[p][c])
                                acc = term if acc is None else acc + term
                            plsc.addupdate(
                                acc_s.at[b, pl.ds(cbase + c * 16, 16)], acc)

        one = jnp.full((16,), 1.0, jnp.float32)
        mone = jnp.full((16,), -1.0, jnp.float32)

        @pl.loop(0, BATCH)
        def _(b):
            for c in range(CPW // 16):
                v = acc_s[b, pl.ds(c * 16, 16)]
                acc_s[b, pl.ds(c * 16, 16)] = jnp.where(v > 0.0, one, mone)

        pltpu.sync_copy(acc_s, out_hbm.at[w])

    return enc(img_r, vt_r, xt_r, yt_r)


def _tc_encode(img_flat, vt_hi, vt_lo, xt, yt):
    nblk = DTC // DBLK

    def body(img_ref, hi_ref, lo_ref, xt_ref, yt_ref, out_ref, acc_ref):
        lane_iota = lax.broadcasted_iota(jnp.int32, (SIZE * BATCH, LEVELS), 1)

        def y_step(y, _):
            onehot = (img_ref[y] == lane_iota).astype(jnp.bfloat16)

            def d_step(j, _):
                hi = hi_ref[:, pl.ds(j * DBLK, DBLK)]
                lo = lo_ref[:, pl.ds(j * DBLK, DBLK)]
                v = jnp.dot(onehot, hi, preferred_element_type=jnp.float32)
                v = v + jnp.dot(onehot, lo, preferred_element_type=jnp.float32)
                w = (xt_ref[:, pl.ds(j * DBLK, DBLK)]
                     * yt_ref[y, pl.ds(j * DBLK, DBLK)][None, :])
                part = (v.reshape(BATCH, SIZE, DBLK) * w[None]).sum(axis=1)
                acc_ref[:, pl.ds(j * DBLK, DBLK)] += part
                return 0

            lax.fori_loop(0, nblk, d_step, 0)
            return 0

        acc_ref[...] = jnp.zeros_like(acc_ref)
        lax.fori_loop(0, SIZE, y_step, 0)
        out_ref[...] = jnp.where(acc_ref[...] > 0.0, 1.0, -1.0).astype(
            jnp.float32)

    return pl.pallas_call(
        body,
        out_shape=jax.ShapeDtypeStruct((BATCH, DTC), jnp.float32),
        scratch_shapes=[pltpu.VMEM((BATCH, DTC), jnp.float32)],
        compiler_params=pltpu.CompilerParams(
            vmem_limit_bytes=100 * 1024 * 1024),
    )(img_flat, vt_hi, vt_lo, xt, yt)


def kernel(x, value_table, x_table, y_table):
    img = x.reshape(BATCH, SIZE, SIZE).astype(jnp.int32)
    img_r = img.transpose(1, 0, 2)
    img_flat = img_r.reshape(SIZE, BATCH * SIZE, 1)

    pad = DPAD - DIM
    vt = jnp.pad(value_table, ((0, 0), (0, pad)))
    xt = jnp.pad(x_table, ((0, 0), (0, pad)))
    yt = jnp.pad(y_table, ((0, 0), (0, pad)))

    vt_sc = vt[:, :DSC].reshape(LEVELS, NW, CPW).transpose(1, 0, 2)
    xt_sc = xt[:, :DSC].reshape(SIZE, NW, CPW).transpose(1, 0, 2)
    yt_sc = yt[:, :DSC].reshape(SIZE, NW, CPW).transpose(1, 0, 2)

    vt_tc = vt[:, DSC:]
    hi_f32 = lax.bitcast_convert_type(
        lax.bitcast_convert_type(vt_tc, jnp.uint32) & jnp.uint32(0xFFFF0000),
        jnp.float32)
    vt_hi = hi_f32.astype(jnp.bfloat16)
    vt_lo = (vt_tc - hi_f32).astype(jnp.bfloat16)

    sc_out = _sc_encode(img_r, vt_sc, xt_sc, yt_sc)
    tc_out = _tc_encode(img_flat, vt_hi, vt_lo, xt[:, DSC:], yt[:, DSC:])

    sc_flat = sc_out.transpose(1, 0, 2).reshape(BATCH, DSC)
    return jnp.concatenate([sc_flat, tc_out], axis=1)[:, :DIM]


# speedup vs baseline: 1.9177x; 1.0464x over previous
import functools

import jax
import jax.numpy as jnp
from jax import lax
from jax.experimental import pallas as pl
from jax.experimental.pallas import tpu as pltpu
from jax.experimental.pallas import tpu_sc as plsc

DIM = 10000
SIZE = 32
LEVELS = 256
BATCH = 64
DPAD = 10240

NW = 32
CPW = 160
DSC = NW * CPW
XG = 16
NXG = SIZE // XG
NCP = 2
CPASS = CPW // (16 * NCP)

DTC = DPAD - DSC
DBLK = 512


def _sc_encode(img_r, vt_r, xt_r, yt_r):
    mesh = plsc.VectorSubcoreMesh(core_axis_name="c", subcore_axis_name="s")

    @functools.partial(
        pl.kernel,
        mesh=mesh,
        out_type=jax.ShapeDtypeStruct((NW, BATCH, CPW), jnp.float32),
        scratch_types=[
            pltpu.VMEM((LEVELS, CPW), jnp.float32),
            pltpu.VMEM((SIZE, CPW), jnp.float32),
            pltpu.VMEM((CPW,), jnp.float32),
            pltpu.VMEM((BATCH, CPW), jnp.float32),
            pltpu.VMEM((BATCH, SIZE), jnp.int32),
            pltpu.SemaphoreType.DMA,
            pltpu.SemaphoreType.DMA,
        ],
        compiler_params=pltpu.CompilerParams(use_tc_tiling_on_sc=False),
    )
    def enc(img_hbm, vt_hbm, xt_hbm, yt_hbm, out_hbm,
            vt_s, xt_s, yt_row, acc_s, idx_s, sem1, sem2):
        w = lax.axis_index("s") * 2 + lax.axis_index("c")
        cp1 = pltpu.async_copy(vt_hbm.at[:, pl.ds(w * CPW, CPW)], vt_s, sem1)
        cp2 = pltpu.async_copy(xt_hbm.at[:, pl.ds(w * CPW, CPW)], xt_s, sem2)

        zero = jnp.zeros((16,), jnp.float32)

        @pl.loop(0, BATCH)
        def _(b):
            for c in range(CPW // 16):
                acc_s[b, pl.ds(c * 16, 16)] = zero

        cp1.wait()
        cp2.wait()

        @pl.loop(0, SIZE)
        def _(y):
            pltpu.sync_copy(img_hbm.at[y], idx_s)
            pltpu.sync_copy(yt_hbm.at[y, pl.ds(w * CPW, CPW)], yt_row)

            @pl.loop(0, CPASS)
            def _(cp):
                cbase = cp * (16 * NCP)
                for xg in range(NXG):
                    wv = [[xt_s[xg * XG + p, pl.ds(cbase + c * 16, 16)]
                           * yt_row[pl.ds(cbase + c * 16, 16)]
                           for c in range(NCP)]
                          for p in range(XG)]

                    @plsc.parallel_loop(0, BATCH)
                    def _(b):
                        lvec = idx_s[b, pl.ds(xg * XG, 16)]
                        for c in range(NCP):
                            acc = None
                            for p in range(XG):
                                term = (vt_s[lvec[p],
                                             pl.ds(cbase + c * 16, 16)]
                                        * wv[p][c])
                                acc = term if acc is None else acc + term
                            plsc.addupdate(
                                acc_s.at[b, pl.ds(cbase + c * 16, 16)], acc)

        one = jnp.full((16,), 1.0, jnp.float32)
        mone = jnp.full((16,), -1.0, jnp.float32)

        @pl.loop(0, BATCH)
        def _(b):
            for c in range(CPW // 16):
                v = acc_s[b, pl.ds(c * 16, 16)]
                acc_s[b, pl.ds(c * 16, 16)] = jnp.where(v > 0.0, one, mone)

        pltpu.sync_copy(acc_s, out_hbm.at[w])

    return enc(img_r, vt_r, xt_r, yt_r)


def _tc_encode(img_flat, vt_hi, vt_lo, xt, yt):
    nblk = DTC // DBLK

    def body(img_ref, hi_ref, lo_ref, xt_ref, yt_ref, out_ref, acc_ref):
        lane_iota = lax.broadcasted_iota(jnp.int32, (SIZE * BATCH, LEVELS), 1)

        def y_step(y, _):
            onehot = (img_ref[y] == lane_iota).astype(jnp.bfloat16)

            def d_step(j, _):
                hi = hi_ref[:, pl.ds(j * DBLK, DBLK)]
                lo = lo_ref[:, pl.ds(j * DBLK, DBLK)]
                v = jnp.dot(onehot, hi, preferred_element_type=jnp.float32)
                v = v + jnp.dot(onehot, lo, preferred_element_type=jnp.float32)
                w = (xt_ref[:, pl.ds(j * DBLK, DBLK)]
                     * yt_ref[y, pl.ds(j * DBLK, DBLK)][None, :])
                part = (v.reshape(BATCH, SIZE, DBLK) * w[None]).sum(axis=1)
                acc_ref[:, pl.ds(j * DBLK, DBLK)] += part
                return 0

            lax.fori_loop(0, nblk, d_step, 0)
            return 0

        acc_ref[...] = jnp.zeros_like(acc_ref)
        lax.fori_loop(0, SIZE, y_step, 0)
        out_ref[...] = jnp.where(acc_ref[...] > 0.0, 1.0, -1.0).astype(
            jnp.float32)

    return pl.pallas_call(
        body,
        out_shape=jax.ShapeDtypeStruct((BATCH, DTC), jnp.float32),
        scratch_shapes=[pltpu.VMEM((BATCH, DTC), jnp.float32)],
        compiler_params=pltpu.CompilerParams(
            vmem_limit_bytes=100 * 1024 * 1024),
    )(img_flat, vt_hi, vt_lo, xt, yt)


def kernel(x, value_table, x_table, y_table):
    img = x.reshape(BATCH, SIZE, SIZE).astype(jnp.int32)
    img_r = img.transpose(1, 0, 2)
    img_flat = img_r.reshape(SIZE, BATCH * SIZE, 1)


    pad = DPAD - DIM
    vt_tc = jnp.pad(value_table[:, DSC:], ((0, 0), (0, pad)))
    xt_tc = jnp.pad(x_table[:, DSC:], ((0, 0), (0, pad)))
    yt_tc = jnp.pad(y_table[:, DSC:], ((0, 0), (0, pad)))
    hi_f32 = lax.bitcast_convert_type(
        lax.bitcast_convert_type(vt_tc, jnp.uint32) & jnp.uint32(0xFFFF0000),
        jnp.float32)
    vt_hi = hi_f32.astype(jnp.bfloat16)
    vt_lo = (vt_tc - hi_f32).astype(jnp.bfloat16)

    sc_out = _sc_encode(img_r, value_table, x_table, y_table)
    tc_out = _tc_encode(img_flat, vt_hi, vt_lo, xt_tc, yt_tc)

    sc_flat = sc_out.transpose(1, 0, 2).reshape(BATCH, DSC)
    return jnp.concatenate([sc_flat, tc_out], axis=1)[:, :DIM]


# speedup vs baseline: 1.9696x; 1.0271x over previous
import functools

import jax
import jax.numpy as jnp
from jax import lax
from jax.experimental import pallas as pl
from jax.experimental.pallas import tpu as pltpu
from jax.experimental.pallas import tpu_sc as plsc

DIM = 10000
SIZE = 32
LEVELS = 256
BATCH = 64
DPAD = 10240

NW = 32
CPW = 160
DSC = NW * CPW
XG = 16
NXG = SIZE // XG
NCP = 2
CPASS = CPW // (16 * NCP)

DTC = DPAD - DSC
DBLK = 512


def _sc_encode(img_r, vt_r, xt_r, yt_r):
    mesh = plsc.VectorSubcoreMesh(core_axis_name="c", subcore_axis_name="s")

    @functools.partial(
        pl.kernel,
        mesh=mesh,
        out_type=jax.ShapeDtypeStruct((NW, BATCH, CPW), jnp.float32),
        scratch_types=[
            pltpu.VMEM((LEVELS, CPW), jnp.float32),
            pltpu.VMEM((SIZE, CPW), jnp.float32),
            pltpu.VMEM((CPW,), jnp.float32),
            pltpu.VMEM((BATCH, CPW), jnp.float32),
            pltpu.VMEM((BATCH, SIZE), jnp.int32),
            pltpu.SemaphoreType.DMA,
            pltpu.SemaphoreType.DMA,
        ],
        compiler_params=pltpu.CompilerParams(use_tc_tiling_on_sc=False),
    )
    def enc(img_hbm, vt_hbm, xt_hbm, yt_hbm, out_hbm,
            vt_s, xt_s, yt_row, acc_s, idx_s, sem1, sem2):
        w = lax.axis_index("s") * 2 + lax.axis_index("c")
        cp1 = pltpu.async_copy(vt_hbm.at[:, pl.ds(w * CPW, CPW)], vt_s, sem1)
        cp2 = pltpu.async_copy(xt_hbm.at[:, pl.ds(w * CPW, CPW)], xt_s, sem2)

        zero = jnp.zeros((16,), jnp.float32)

        @pl.loop(0, BATCH)
        def _(b):
            for c in range(CPW // 16):
                acc_s[b, pl.ds(c * 16, 16)] = zero

        cp1.wait()
        cp2.wait()

        @pl.loop(0, SIZE)
        def _(y):
            pltpu.sync_copy(img_hbm.at[y], idx_s)
            pltpu.sync_copy(yt_hbm.at[y, pl.ds(w * CPW, CPW)], yt_row)

            @pl.loop(0, CPASS)
            def _(cp):
                cbase = cp * (16 * NCP)
                for xg in range(NXG):
                    wv = [[xt_s[xg * XG + p, pl.ds(cbase + c * 16, 16)]
                           * yt_row[pl.ds(cbase + c * 16, 16)]
                           for c in range(NCP)]
                          for p in range(XG)]

                    @plsc.parallel_loop(0, BATCH)
                    def _(b):
                        lvec = idx_s[b, pl.ds(xg * XG, 16)]
                        for c in range(NCP):
                            acc = None
                            for p in range(XG):
                                term = (vt_s[lvec[p],
                                             pl.ds(cbase + c * 16, 16)]
                                        * wv[p][c])
                                acc = term if acc is None else acc + term
                            plsc.addupdate(
                                acc_s.at[b, pl.ds(cbase + c * 16, 16)], acc)

        one = jnp.full((16,), 1.0, jnp.float32)
        mone = jnp.full((16,), -1.0, jnp.float32)

        @pl.loop(0, BATCH)
        def _(b):
            for c in range(CPW // 16):
                v = acc_s[b, pl.ds(c * 16, 16)]
                acc_s[b, pl.ds(c * 16, 16)] = jnp.where(v > 0.0, one, mone)

        pltpu.sync_copy(acc_s, out_hbm.at[w])

    return enc(img_r, vt_r, xt_r, yt_r)


def _tc_encode(img_flat, vt_hi, vt_lo, xt, yt):
    nblk = DTC // DBLK

    def body(img_ref, hi_ref, lo_ref, xt_ref, yt_ref, out_ref, acc_ref):
        lane_iota = lax.broadcasted_iota(jnp.int32, (SIZE * BATCH, LEVELS), 1)

        def y_step(y, _):
            onehot = (img_ref[y] == lane_iota).astype(jnp.bfloat16)

            def d_step(j, _):
                hi = hi_ref[:, pl.ds(j * DBLK, DBLK)]
                lo = lo_ref[:, pl.ds(j * DBLK, DBLK)]
                v = jnp.dot(onehot, hi, preferred_element_type=jnp.float32)
                v = v + jnp.dot(onehot, lo, preferred_element_type=jnp.float32)
                w = (xt_ref[:, pl.ds(j * DBLK, DBLK)]
                     * yt_ref[y, pl.ds(j * DBLK, DBLK)][None, :])
                part = (v.reshape(SIZE, BATCH, DBLK) * w[:, None, :]).sum(axis=0)
                acc_ref[:, pl.ds(j * DBLK, DBLK)] += part
                return 0

            lax.fori_loop(0, nblk, d_step, 0)
            return 0

        acc_ref[...] = jnp.zeros_like(acc_ref)
        lax.fori_loop(0, SIZE, y_step, 0)
        out_ref[...] = jnp.where(acc_ref[...] > 0.0, 1.0, -1.0).astype(
            jnp.float32)

    return pl.pallas_call(
        body,
        out_shape=jax.ShapeDtypeStruct((BATCH, DTC), jnp.float32),
        scratch_shapes=[pltpu.VMEM((BATCH, DTC), jnp.float32)],
        compiler_params=pltpu.CompilerParams(
            vmem_limit_bytes=100 * 1024 * 1024),
    )(img_flat, vt_hi, vt_lo, xt, yt)


def kernel(x, value_table, x_table, y_table):
    img = x.reshape(BATCH, SIZE, SIZE).astype(jnp.int32)
    img_r = img.transpose(1, 0, 2)
    img_flat = img.transpose(1, 2, 0).reshape(SIZE, SIZE * BATCH, 1)


    pad = DPAD - DIM
    vt_tc = jnp.pad(value_table[:, DSC:], ((0, 0), (0, pad)))
    xt_tc = jnp.pad(x_table[:, DSC:], ((0, 0), (0, pad)))
    yt_tc = jnp.pad(y_table[:, DSC:], ((0, 0), (0, pad)))
    hi_f32 = lax.bitcast_convert_type(
        lax.bitcast_convert_type(vt_tc, jnp.uint32) & jnp.uint32(0xFFFF0000),
        jnp.float32)
    vt_hi = hi_f32.astype(jnp.bfloat16)
    vt_lo = (vt_tc - hi_f32).astype(jnp.bfloat16)

    sc_out = _sc_encode(img_r, value_table, x_table, y_table)
    tc_out = _tc_encode(img_flat, vt_hi, vt_lo, xt_tc, yt_tc)

    sc_flat = sc_out.transpose(1, 0, 2).reshape(BATCH, DSC)
    return jnp.concatenate([sc_flat, tc_out], axis=1)[:, :DIM]
